# Initial kernel scaffold; baseline (speedup 1.0000x reference)
import jax
import jax.numpy as jnp
from jax.experimental import pallas as pl


def kernel(x, edge_index, batch, W1, b1, W2, b2):
    raise NotImplementedError("write your pallas kernel here")


# speedup vs baseline: 15.5240x; 15.5240x over previous
import jax
import jax.numpy as jnp
from jax.experimental import pallas as pl

NUM_GRAPHS = 64


def _tc_body(x_ref, batch_ref, W1_ref, b1_ref, W2_ref, b2_ref, out_ref):
    batch = batch_ref[0, :]
    n = batch.shape[0]
    gids = jax.lax.broadcasted_iota(jnp.int32, (NUM_GRAPHS, n), 0)
    onehot = (batch[None, :] == gids).astype(jnp.float32)
    sums = jnp.dot(onehot, x_ref[...], preferred_element_type=jnp.float32)
    counts = jnp.sum(onehot, axis=1, keepdims=True)
    pooled = sums / jnp.maximum(counts, 1.0)
    h = jnp.maximum(
        jnp.dot(pooled, W1_ref[...], preferred_element_type=jnp.float32)
        + b1_ref[...],
        0.0,
    )
    out_ref[...] = (
        jnp.dot(h, W2_ref[...], preferred_element_type=jnp.float32) + b2_ref[...]
    )


def kernel(x, edge_index, batch, W1, b1, W2, b2):
    del edge_index
    batch2 = batch.reshape(1, -1)
    b1r = b1.reshape(1, -1)
    b2r = b2.reshape(1, -1)
    out = pl.pallas_call(
        _tc_body,
        out_shape=jax.ShapeDtypeStruct((NUM_GRAPHS, W2.shape[1]), jnp.float32),
    )(x, batch2, W1, b1r, W2, b2r)
    return out
